# Initial kernel scaffold; baseline (speedup 1.0000x reference)
import jax
import jax.numpy as jnp
from jax.experimental import pallas as pl


def kernel(x, tok_table, pos_emb):
    raise NotImplementedError("write your pallas kernel here")


# speedup vs baseline: 2.0077x; 2.0077x over previous
import functools

import jax
import jax.numpy as jnp
from jax import lax
from jax.experimental import pallas as pl
from jax.experimental.pallas import tpu as pltpu
from jax.experimental.pallas import tpu_sc as plsc

B, T, D = 1024, 200, 64
NC, NS = 2, 16
NW = NC * NS
RPW = B // NW
LANES = 16
VECS_PER_ROW = D // LANES

_mesh = plsc.VectorSubcoreMesh(core_axis_name="c", subcore_axis_name="s")


@functools.partial(
    pl.kernel,
    out_type=jax.ShapeDtypeStruct((B, T, D), jnp.float32),
    mesh=_mesh,
    compiler_params=pltpu.CompilerParams(use_tc_tiling_on_sc=False),
    scratch_types=[
        pltpu.VMEM((T,), jnp.int32),
        pltpu.VMEM((T, D), jnp.float32),
        pltpu.VMEM((T, D), jnp.float32),
        pltpu.SemaphoreType.DMA,
    ],
)
def _embed(x_hbm, tab_hbm, pos_hbm, out_hbm, idx_v, rows_v, pos_v, sem):
    wid = lax.axis_index("s") * NC + lax.axis_index("c")
    pltpu.sync_copy(pos_hbm.at[pl.ds(0, T)], pos_v)

    def row_body(r, carry):
        b = wid * RPW + r
        pltpu.sync_copy(x_hbm.at[b], idx_v)
        c0 = pltpu.async_copy(
            tab_hbm.at[idx_v.at[pl.ds(0, 128)]], rows_v.at[pl.ds(0, 128)], sem)
        c1 = pltpu.async_copy(
            tab_hbm.at[idx_v.at[pl.ds(128, 72)]], rows_v.at[pl.ds(128, 72)], sem)
        c0.wait()
        c1.wait()

        def add_body(t, c2):
            for c in range(VECS_PER_ROW):
                sl = (t, pl.ds(c * LANES, LANES))
                rows_v[sl] = rows_v[sl] + pos_v[sl]
            return c2

        lax.fori_loop(0, T, add_body, 0, unroll=2)
        pltpu.sync_copy(rows_v, out_hbm.at[b])
        return carry

    lax.fori_loop(0, RPW, row_body, 0)


def kernel(x, tok_table, pos_emb):
    return _embed(x.astype(jnp.int32), tok_table, pos_emb)


# speedup vs baseline: 2.5211x; 1.2557x over previous
import functools

import jax
import jax.numpy as jnp
from jax import lax
from jax.experimental import pallas as pl
from jax.experimental.pallas import tpu as pltpu
from jax.experimental.pallas import tpu_sc as plsc

B, T, D = 1024, 200, 64
NC, NS = 2, 16
NW = NC * NS
RPW = B // NW
LANES = 16
VECS_PER_ROW = D // LANES

_mesh = plsc.VectorSubcoreMesh(core_axis_name="c", subcore_axis_name="s")


@functools.partial(
    pl.kernel,
    out_type=jax.ShapeDtypeStruct((B, T, D), jnp.float32),
    mesh=_mesh,
    compiler_params=pltpu.CompilerParams(use_tc_tiling_on_sc=False),
    scratch_types=[
        pltpu.VMEM((RPW, T), jnp.int32),
        pltpu.VMEM((2, T, D), jnp.float32),
        pltpu.VMEM((T, D), jnp.float32),
        pltpu.SemaphoreType.DMA,
        pltpu.SemaphoreType.DMA,
    ],
)
def _embed(x_hbm, tab_hbm, pos_hbm, out_hbm, idx_v, rows_v, pos_v,
           gsem, osem):
    wid = lax.axis_index("s") * NC + lax.axis_index("c")
    base = wid * RPW
    pltpu.sync_copy(pos_hbm.at[pl.ds(0, T)], pos_v)
    pltpu.sync_copy(x_hbm.at[pl.ds(base, RPW)], idx_v)

    def start_gathers(r, buf):
        pltpu.async_copy(
            tab_hbm.at[idx_v.at[r, pl.ds(0, 128)]],
            rows_v.at[buf, pl.ds(0, 128)], gsem)
        pltpu.async_copy(
            tab_hbm.at[idx_v.at[r, pl.ds(128, 72)]],
            rows_v.at[buf, pl.ds(128, 72)], gsem)

    def wait_gathers(r, buf):
        pltpu.make_async_copy(
            tab_hbm.at[idx_v.at[r, pl.ds(0, 128)]],
            rows_v.at[buf, pl.ds(0, 128)], gsem).wait()
        pltpu.make_async_copy(
            tab_hbm.at[idx_v.at[r, pl.ds(128, 72)]],
            rows_v.at[buf, pl.ds(128, 72)], gsem).wait()

    start_gathers(0, 0)

    def row_body(r, carry):
        buf = lax.rem(r, 2)
        nbuf = lax.rem(r + 1, 2)

        @pl.when(r >= 1)
        def _():
            pltpu.make_async_copy(
                rows_v.at[nbuf], out_hbm.at[base + r - 1], osem).wait()

        @pl.when(r + 1 < RPW)
        def _():
            start_gathers(r + 1, nbuf)

        wait_gathers(r, buf)

        def add_body(t, c2):
            for c in range(VECS_PER_ROW):
                plsc.addupdate(
                    rows_v.at[buf, t, pl.ds(c * LANES, LANES)],
                    pos_v[t, pl.ds(c * LANES, LANES)])
            return c2

        lax.fori_loop(0, T, add_body, 0, unroll=4)

        pltpu.async_copy(rows_v.at[buf], out_hbm.at[base + r], osem)
        return carry

    lax.fori_loop(0, RPW, row_body, 0)
    pltpu.make_async_copy(
        rows_v.at[(RPW - 1) % 2], out_hbm.at[base + RPW - 1], osem).wait()


def kernel(x, tok_table, pos_emb):
    return _embed(x.astype(jnp.int32), tok_table, pos_emb)
